# baseline (speedup 1.0000x reference)
import functools

import jax
import jax.numpy as jnp
from jax import lax
from jax.experimental import pallas as pl
from jax.experimental.pallas import tpu as pltpu
from jax.experimental.pallas import tpu_sc as plsc

VOCAB = 1002
DIM = 128
WIN = 128
NWORKERS = 32
NBUF = 4


def kernel(table, indices_tensor):
    batch, seq = indices_tensor.shape
    n = batch * seq
    nwin = n // WIN
    wpw = nwin // NWORKERS
    idx2d = indices_tensor.reshape(nwin, WIN).astype(jnp.int32)

    mesh = plsc.VectorSubcoreMesh(core_axis_name="c", subcore_axis_name="s")

    @functools.partial(
        pl.kernel,
        out_type=jax.ShapeDtypeStruct((n, DIM), table.dtype),
        mesh=mesh,
        scratch_types=[
            pltpu.VMEM_SHARED((VOCAB, DIM), jnp.float32),
            pltpu.VMEM((wpw, WIN), jnp.int32),
            pltpu.VMEM((NBUF, WIN, DIM), jnp.float32),
            pltpu.SemaphoreType.DMA,
            pltpu.SemaphoreType.DMA,
        ],
    )
    def gather_kernel(table_hbm, idx_hbm, out_hbm, table_sh, idx_v, bufs,
                      isem, wsem):
        cid = lax.axis_index("c")
        sid = lax.axis_index("s")
        wid = sid * 2 + cid

        idx_cp = pltpu.async_copy(idx_hbm.at[pl.ds(wid * wpw, wpw)], idx_v, isem)

        @pl.when(sid == 0)
        def _():
            pltpu.sync_copy(table_hbm, table_sh)

        idx_cp.wait()
        plsc.subcore_barrier()

        base = wid * (wpw * WIN)

        @pl.loop(0, wpw, step=NBUF)
        def _(j0):
            for b in range(NBUF):
                j = j0 + b
                slot = bufs.at[b]

                @pl.when(j0 > 0)
                def _():
                    pltpu.make_async_copy(
                        slot, out_hbm.at[pl.ds(base, WIN)], wsem).wait()

                pltpu.sync_copy(table_sh.at[idx_v.at[j]], slot)
                pltpu.async_copy(
                    slot, out_hbm.at[pl.ds(base + j * WIN, WIN)], wsem)

        for b in range(NBUF):
            pltpu.make_async_copy(
                bufs.at[b], out_hbm.at[pl.ds(base, WIN)], wsem).wait()

    out = gather_kernel(table, idx2d)
    return out.reshape(batch, seq, DIM)


# speedup vs baseline: 1.0705x; 1.0705x over previous
import functools

import jax
import jax.numpy as jnp
from jax import lax
from jax.experimental import pallas as pl
from jax.experimental.pallas import tpu as pltpu
from jax.experimental.pallas import tpu_sc as plsc

VOCAB = 1002
DIM = 128
WIN = 128
NWORKERS = 32
NBUF = 4


def kernel(table, indices_tensor):
    batch, seq = indices_tensor.shape
    n = batch * seq
    nwin = n // WIN
    wpw = nwin // NWORKERS
    idx2d = indices_tensor.reshape(nwin, WIN).astype(jnp.int32)

    mesh = plsc.VectorSubcoreMesh(core_axis_name="c", subcore_axis_name="s")

    @functools.partial(
        pl.kernel,
        out_type=jax.ShapeDtypeStruct((n, DIM), table.dtype),
        mesh=mesh,
        scratch_types=[
            pltpu.VMEM_SHARED((VOCAB, DIM), jnp.float32),
            pltpu.VMEM((wpw, WIN), jnp.int32),
            pltpu.VMEM((NBUF, WIN, DIM), jnp.float32),
            pltpu.SemaphoreType.DMA,
            pltpu.SemaphoreType.DMA,
        ],
    )
    def gather_kernel(table_hbm, idx_hbm, out_hbm, table_sh, idx_v, bufs,
                      gsem, wsem):
        cid = lax.axis_index("c")
        sid = lax.axis_index("s")
        wid = sid * 2 + cid

        idx_cp = pltpu.async_copy(idx_hbm.at[pl.ds(wid * wpw, wpw)], idx_v, gsem)

        @pl.when(sid == 0)
        def _():
            pltpu.sync_copy(table_hbm, table_sh)

        idx_cp.wait()
        plsc.subcore_barrier()

        base = wid * (wpw * WIN)

        @pl.loop(0, wpw, step=NBUF)
        def _(j0):
            gathers = []
            for b in range(NBUF):
                slot = bufs.at[b]

                @pl.when(j0 > 0)
                def _():
                    pltpu.make_async_copy(
                        slot, out_hbm.at[pl.ds(base, WIN)], wsem).wait()

                gathers.append(pltpu.async_copy(
                    table_sh.at[idx_v.at[j0 + b]], slot, gsem))

            for b in range(NBUF):
                gathers[b].wait()
                pltpu.async_copy(
                    bufs.at[b], out_hbm.at[pl.ds(base + (j0 + b) * WIN, WIN)],
                    wsem)

        for b in range(NBUF):
            pltpu.make_async_copy(
                bufs.at[b], out_hbm.at[pl.ds(base, WIN)], wsem).wait()

    out = gather_kernel(table, idx2d)
    return out.reshape(batch, seq, DIM)
